# Initial kernel scaffold; baseline (speedup 1.0000x reference)
import jax
import jax.numpy as jnp
from jax.experimental import pallas as pl


def kernel(x, edge_index, W, b):
    raise NotImplementedError("write your pallas kernel here")


# speedup vs baseline: 12.0107x; 12.0107x over previous
import functools

import jax
import jax.numpy as jnp
from jax import lax
from jax.experimental import pallas as pl
from jax.experimental.pallas import tpu as pltpu
from jax.experimental.pallas import tpu_sc as plsc

N = 10000
E = 320000
D = 128

NC = 2
NS = 16
CH = 128
NP = 10240
RPT = NP // NS

EPT = ((E + NC * NS * CH * 8 - 1) // (NC * NS * CH * 8)) * CH * 8
JCH = EPT // CH
EPAD = EPT * NC * NS

_mesh = plsc.VectorSubcoreMesh(core_axis_name="c", subcore_axis_name="s")


def _zero_fill(vref, rows, width):
    @pl.loop(0, rows)
    def _(i):
        @pl.loop(0, width, step=16)
        def _(j):
            vref[i, pl.ds(j, 16)] = jnp.zeros((16,), jnp.float32)


def _zero_shared(zsrc, acc_sh, base, width):
    @pl.loop(0, RPT // CH)
    def _(t):
        pltpu.sync_copy(zsrc, acc_sh.at[pl.ds(base + t * CH, CH)])


@functools.partial(
    pl.kernel,
    out_type=jax.ShapeDtypeStruct((NC, NP, D), jnp.float32),
    mesh=_mesh,
    scratch_types=[
        pltpu.VMEM((JCH, CH), jnp.int32),
        pltpu.VMEM((CH, D), jnp.float32),
        pltpu.VMEM_SHARED((NP, D), jnp.float32),
    ],
)
def _deg_kernel(dst_hbm, deg_out, idx_v, ones_v, acc_sh):
    c = lax.axis_index("c")
    s = lax.axis_index("s")
    w = c * NS + s
    base = s * RPT

    _zero_fill(ones_v, CH, D)
    _zero_shared(ones_v, acc_sh, base, D)

    @pl.loop(0, CH)
    def _(i):
        @pl.loop(0, D, step=16)
        def _(j):
            ones_v[i, pl.ds(j, 16)] = jnp.ones((16,), jnp.float32)

    plsc.subcore_barrier()

    pltpu.sync_copy(dst_hbm.at[pl.ds(w * JCH, JCH)], idx_v)

    @pl.loop(0, JCH)
    def _(j):
        pltpu.sync_copy(ones_v, acc_sh.at[idx_v.at[j]], add=True)

    plsc.subcore_barrier()
    pltpu.sync_copy(acc_sh.at[pl.ds(base, RPT)], deg_out.at[c, pl.ds(base, RPT)])


@functools.partial(
    pl.kernel,
    out_type=jax.ShapeDtypeStruct((NC, NP, D), jnp.float32),
    mesh=_mesh,
    scratch_types=[
        pltpu.VMEM((JCH, CH), jnp.int32),
        pltpu.VMEM((JCH, CH), jnp.int32),
        pltpu.VMEM((CH, D), jnp.float32),
        pltpu.VMEM_SHARED((NP, D), jnp.float32),
    ],
)
def _agg_kernel(h2_hbm, src_hbm, dst_hbm, p_out, src_v, dst_v, rows_v, acc_sh):
    c = lax.axis_index("c")
    s = lax.axis_index("s")
    w = c * NS + s
    base = s * RPT

    _zero_fill(rows_v, CH, D)
    _zero_shared(rows_v, acc_sh, base, D)
    plsc.subcore_barrier()

    pltpu.sync_copy(src_hbm.at[pl.ds(w * JCH, JCH)], src_v)
    pltpu.sync_copy(dst_hbm.at[pl.ds(w * JCH, JCH)], dst_v)

    @pl.loop(0, JCH)
    def _(j):
        pltpu.sync_copy(h2_hbm.at[src_v.at[j]], rows_v)
        pltpu.sync_copy(rows_v, acc_sh.at[dst_v.at[j]], add=True)

    plsc.subcore_barrier()
    pltpu.sync_copy(acc_sh.at[pl.ds(base, RPT)], p_out.at[c, pl.ds(base, RPT)])


def _mm_body(x_ref, w_ref, h_ref):
    h_ref[...] = jnp.dot(x_ref[...], w_ref[...], preferred_element_type=jnp.float32)


def _scale_body(h_ref, deg_ref, h2_ref):
    deg = deg_ref[0, :, 0] + deg_ref[1, :, 0]
    isd = lax.rsqrt(jnp.maximum(deg, 1.0))
    h2_ref[...] = h_ref[...] * isd[:, None]


def _final_body(p_ref, deg_ref, b_ref, o_ref):
    deg = deg_ref[0, :, 0] + deg_ref[1, :, 0]
    isd = lax.rsqrt(jnp.maximum(deg, 1.0))
    o_ref[...] = (p_ref[0] + p_ref[1]) * isd[:, None] + b_ref[...][None, :]


_BLK = 1000


@jax.jit
def _impl(x, edge_index, W, b):
    src = edge_index[0]
    dst = edge_index[1]
    pad = EPAD - E
    src_p = jnp.concatenate([src, jnp.zeros((pad,), jnp.int32)]).reshape(-1, CH)
    dst_p = jnp.concatenate([dst, jnp.full((pad,), N, jnp.int32)]).reshape(-1, CH)

    h = pl.pallas_call(
        _mm_body,
        grid=(N // _BLK,),
        in_specs=[
            pl.BlockSpec((_BLK, D), lambda i: (i, 0)),
            pl.BlockSpec((D, D), lambda i: (0, 0)),
        ],
        out_specs=pl.BlockSpec((_BLK, D), lambda i: (i, 0)),
        out_shape=jax.ShapeDtypeStruct((N, D), jnp.float32),
    )(x, W)

    deg16 = _deg_kernel(dst_p)

    h2 = pl.pallas_call(
        _scale_body,
        grid=(N // _BLK,),
        in_specs=[
            pl.BlockSpec((_BLK, D), lambda i: (i, 0)),
            pl.BlockSpec((NC, _BLK, D), lambda i: (0, i, 0)),
        ],
        out_specs=pl.BlockSpec((_BLK, D), lambda i: (i, 0)),
        out_shape=jax.ShapeDtypeStruct((N, D), jnp.float32),
    )(h, deg16)

    parts = _agg_kernel(h2, src_p, dst_p)

    out = pl.pallas_call(
        _final_body,
        grid=(N // _BLK,),
        in_specs=[
            pl.BlockSpec((NC, _BLK, D), lambda i: (0, i, 0)),
            pl.BlockSpec((NC, _BLK, D), lambda i: (0, i, 0)),
            pl.BlockSpec((D,), lambda i: (0,)),
        ],
        out_specs=pl.BlockSpec((_BLK, D), lambda i: (i, 0)),
        out_shape=jax.ShapeDtypeStruct((N, D), jnp.float32),
    )(parts, deg16, b)
    return out


def kernel(x, edge_index, W, b):
    return _impl(x, edge_index, W, b)


# speedup vs baseline: 13.0663x; 1.0879x over previous
import functools

import jax
import jax.numpy as jnp
from jax import lax
from jax.experimental import pallas as pl
from jax.experimental.pallas import tpu as pltpu
from jax.experimental.pallas import tpu_sc as plsc

N = 10000
E = 320000
D = 128

NC = 2
NS = 16
CH = 128
NP = 10240
RPT = NP // NS

EPT = ((E + NC * NS * CH * 8 - 1) // (NC * NS * CH * 8)) * CH * 8
JCH = EPT // CH
EPAD = EPT * NC * NS

_mesh = plsc.VectorSubcoreMesh(core_axis_name="c", subcore_axis_name="s")


def _zero_fill(vref, rows, width):
    @pl.loop(0, rows)
    def _(i):
        @pl.loop(0, width, step=16)
        def _(j):
            vref[i, pl.ds(j, 16)] = jnp.zeros((16,), jnp.float32)


def _zero_shared(zsrc, acc_sh, base, width):
    @pl.loop(0, RPT // CH)
    def _(t):
        pltpu.sync_copy(zsrc, acc_sh.at[pl.ds(base + t * CH, CH)])


@functools.partial(
    pl.kernel,
    out_type=jax.ShapeDtypeStruct((NC, NP, D), jnp.float32),
    mesh=_mesh,
    scratch_types=[
        pltpu.VMEM((JCH, CH), jnp.int32),
        pltpu.VMEM((CH, D), jnp.float32),
        pltpu.VMEM_SHARED((NP, D), jnp.float32),
        pltpu.SemaphoreType.DMA,
    ],
)
def _deg_kernel(dst_hbm, deg_out, idx_v, ones_v, acc_sh, dsem):
    c = lax.axis_index("c")
    s = lax.axis_index("s")
    w = c * NS + s
    base = s * RPT

    _zero_fill(ones_v, CH, D)
    _zero_shared(ones_v, acc_sh, base, D)

    @pl.loop(0, CH)
    def _(i):
        @pl.loop(0, D, step=16)
        def _(j):
            ones_v[i, pl.ds(j, 16)] = jnp.ones((16,), jnp.float32)

    plsc.subcore_barrier()

    pltpu.sync_copy(dst_hbm.at[pl.ds(w * JCH, JCH)], idx_v)

    @pl.loop(0, JCH, step=8)
    def _(j):
        for g in range(8):
            pltpu.async_copy(ones_v, acc_sh.at[idx_v.at[j + g]], dsem, add=True)
        for g in range(8):
            pltpu.make_async_copy(ones_v, acc_sh.at[idx_v.at[j + g]], dsem).wait()

    plsc.subcore_barrier()
    pltpu.sync_copy(acc_sh.at[pl.ds(base, RPT)], deg_out.at[c, pl.ds(base, RPT)])


NBUF = 2
NHALF = 2
HCH = JCH // NHALF
STEPS = HCH // NBUF


@functools.partial(
    pl.kernel,
    out_type=jax.ShapeDtypeStruct((NC, NP, D), jnp.float32),
    mesh=_mesh,
    scratch_types=[
        pltpu.VMEM((HCH, CH), jnp.int32),
        pltpu.VMEM((HCH, CH), jnp.int32),
        [pltpu.VMEM((CH, D), jnp.float32)] * NBUF,
        [pltpu.SemaphoreType.DMA] * NBUF,
        [pltpu.SemaphoreType.DMA] * NBUF,
        pltpu.VMEM_SHARED((NP, D), jnp.float32),
    ],
)
def _agg_kernel(h2_hbm, src_hbm, dst_hbm, p_out, src_v, dst_v, rows, gsem,
                ssem, acc_sh):
    c = lax.axis_index("c")
    s = lax.axis_index("s")
    w = c * NS + s
    base = s * RPT

    _zero_fill(rows[0], CH, D)
    _zero_shared(rows[0], acc_sh, base, D)
    plsc.subcore_barrier()

    for half in range(NHALF):
        pltpu.sync_copy(src_hbm.at[pl.ds(w * JCH + half * HCH, HCH)], src_v)
        pltpu.sync_copy(dst_hbm.at[pl.ds(w * JCH + half * HCH, HCH)], dst_v)

        for b in range(NBUF):
            pltpu.async_copy(h2_hbm.at[src_v.at[b]], rows[b], gsem[b])

        @pl.loop(0, STEPS)
        def _(t):
            j0 = t * NBUF
            for b in range(NBUF):
                pltpu.make_async_copy(h2_hbm.at[src_v.at[j0 + b]], rows[b],
                                      gsem[b]).wait()
                pltpu.async_copy(rows[b], acc_sh.at[dst_v.at[j0 + b]], ssem[b],
                                 add=True)

            @pl.when(t + 1 < STEPS)
            def _():
                for b in range(NBUF):
                    pltpu.make_async_copy(rows[b], acc_sh.at[dst_v.at[j0 + b]],
                                          ssem[b]).wait()
                    pltpu.async_copy(h2_hbm.at[src_v.at[j0 + NBUF + b]],
                                     rows[b], gsem[b])

        for b in range(NBUF):
            pltpu.make_async_copy(rows[b],
                                  acc_sh.at[dst_v.at[(STEPS - 1) * NBUF + b]],
                                  ssem[b]).wait()

    plsc.subcore_barrier()
    pltpu.sync_copy(acc_sh.at[pl.ds(base, RPT)], p_out.at[c, pl.ds(base, RPT)])


def _mm_body(x_ref, w_ref, h_ref):
    h_ref[...] = jnp.dot(x_ref[...], w_ref[...], preferred_element_type=jnp.float32)


def _scale_body(h_ref, deg_ref, h2_ref):
    deg = deg_ref[0, :, 0] + deg_ref[1, :, 0]
    isd = lax.rsqrt(jnp.maximum(deg, 1.0))
    h2_ref[...] = h_ref[...] * isd[:, None]


def _final_body(p_ref, deg_ref, b_ref, o_ref):
    deg = deg_ref[0, :, 0] + deg_ref[1, :, 0]
    isd = lax.rsqrt(jnp.maximum(deg, 1.0))
    o_ref[...] = (p_ref[0] + p_ref[1]) * isd[:, None] + b_ref[...][None, :]


_BLK = 1000


@jax.jit
def _impl(x, edge_index, W, b):
    src = edge_index[0]
    dst = edge_index[1]
    pad = EPAD - E
    src_p = jnp.concatenate([src, jnp.zeros((pad,), jnp.int32)]).reshape(-1, CH)
    dst_p = jnp.concatenate([dst, jnp.full((pad,), N, jnp.int32)]).reshape(-1, CH)

    h = pl.pallas_call(
        _mm_body,
        grid=(N // _BLK,),
        in_specs=[
            pl.BlockSpec((_BLK, D), lambda i: (i, 0)),
            pl.BlockSpec((D, D), lambda i: (0, 0)),
        ],
        out_specs=pl.BlockSpec((_BLK, D), lambda i: (i, 0)),
        out_shape=jax.ShapeDtypeStruct((N, D), jnp.float32),
    )(x, W)

    deg16 = _deg_kernel(dst_p)

    h2 = pl.pallas_call(
        _scale_body,
        grid=(N // _BLK,),
        in_specs=[
            pl.BlockSpec((_BLK, D), lambda i: (i, 0)),
            pl.BlockSpec((NC, _BLK, D), lambda i: (0, i, 0)),
        ],
        out_specs=pl.BlockSpec((_BLK, D), lambda i: (i, 0)),
        out_shape=jax.ShapeDtypeStruct((N, D), jnp.float32),
    )(h, deg16)

    parts = _agg_kernel(h2, src_p, dst_p)

    out = pl.pallas_call(
        _final_body,
        grid=(N // _BLK,),
        in_specs=[
            pl.BlockSpec((NC, _BLK, D), lambda i: (0, i, 0)),
            pl.BlockSpec((NC, _BLK, D), lambda i: (0, i, 0)),
            pl.BlockSpec((D,), lambda i: (0,)),
        ],
        out_specs=pl.BlockSpec((_BLK, D), lambda i: (i, 0)),
        out_shape=jax.ShapeDtypeStruct((N, D), jnp.float32),
    )(parts, deg16, b)
    return out


def kernel(x, edge_index, W, b):
    return _impl(x, edge_index, W, b)
